# baseline (speedup 1.0000x reference)
import functools

import jax
import jax.numpy as jnp
from jax import lax
from jax.experimental import pallas as pl
from jax.experimental.pallas import tpu as pltpu
from jax.experimental.pallas import tpu_sc as plsc

VOCAB = 1000000
EMBED_DIM = 32
BATCH = 16384
FIELDS = 26

NUM_CORES = 2
NUM_SUBCORES = 16
NUM_WORKERS = NUM_CORES * NUM_SUBCORES

CONV_COLS = 256
CONV_UNITS = (VOCAB - 64) // CONV_COLS
CONV_ITERS = (CONV_UNITS + NUM_WORKERS - 1) // NUM_WORKERS
CONV_LINES = CONV_COLS // 4
SCR_LINES = ((VOCAB + 127) // 128) * 32

BBLOCKS = BATCH // 128
UNITS = FIELDS * BBLOCKS
UPW = UNITS // NUM_WORKERS

_MESH = plsc.VectorSubcoreMesh(core_axis_name="c", subcore_axis_name="s")
_PARAMS = pltpu.CompilerParams(needs_layout_passes=False)


@functools.partial(
    pl.kernel,
    out_type=jax.ShapeDtypeStruct((SCR_LINES, 128), jnp.float32),
    mesh=_MESH,
    scratch_types=[
        [pltpu.VMEM((EMBED_DIM, CONV_COLS + 1), jnp.float32) for _ in range(2)],
        [pltpu.VMEM((CONV_LINES, 128), jnp.float32) for _ in range(2)],
        pltpu.VMEM((64, 32), jnp.float32),
        [pltpu.SemaphoreType.DMA for _ in range(2)],
        [pltpu.SemaphoreType.DMA for _ in range(2)],
    ],
    compiler_params=_PARAMS,
)
def _convert(emb_t, emb_tail, scr, src, lines, tail_v, isem, osem):
    wid = lax.axis_index("s") * NUM_CORES + lax.axis_index("c")
    iota16 = lax.iota(jnp.int32, 16)

    def unit_of(it):
        return wid + NUM_WORKERS * it

    def ok(it):
        return unit_of(it) < CONV_UNITS

    def start_in(it, b):
        base = pl.multiple_of(unit_of(it) * CONV_COLS, CONV_COLS)
        pltpu.async_copy(
            emb_t.at[:, pl.ds(base, CONV_COLS)],
            src[b].at[:, pl.ds(0, CONV_COLS)],
            isem[b],
        )

    def wait_in(b):
        pltpu.make_async_copy(
            emb_t.at[:, pl.ds(0, CONV_COLS)],
            src[b].at[:, pl.ds(0, CONV_COLS)],
            isem[b],
        ).wait()

    def start_out(it, b):
        lbase = pl.multiple_of(unit_of(it) * CONV_LINES, CONV_LINES)
        pltpu.async_copy(lines[b], scr.at[pl.ds(lbase, CONV_LINES), :], osem[b])

    def wait_out(b):
        pltpu.make_async_copy(
            lines[b], scr.at[pl.ds(0, CONV_LINES), :], osem[b]
        ).wait()

    e_vecs = (iota16, iota16 + 16)

    def assemble(b):
        def jblock(jj, carry):
            for dj in range(8):
                j = jj * 8 + dj
                for g in range(8):
                    col = jnp.full((16,), 4 * j + g // 2, jnp.int32)
                    lines[b][j, pl.ds(16 * g, 16)] = plsc.load_gather(
                        src[b], [e_vecs[g % 2], col]
                    )
            return carry

        lax.fori_loop(0, CONV_LINES // 8, jblock, 0)

    @pl.when(ok(0))
    def _():
        start_in(0, 0)

    def body(step, carry):
        for half in range(2):
            it = 2 * step + half
            b = half

            @pl.when(ok(it))
            def _():
                @pl.when(ok(it + 1))
                def _():
                    start_in(it + 1, 1 - b)

                wait_in(b)

                @pl.when(it >= 2)
                def _():
                    wait_out(b)

                assemble(b)
                start_out(it, b)

        return carry

    lax.fori_loop(0, (CONV_ITERS + 1) // 2, body, 0)

    for t in (CONV_ITERS - 3, CONV_ITERS - 2, CONV_ITERS - 1):

        @pl.when(ok(t) & jnp.logical_not(ok(t + 2)))
        def _(t=t):
            wait_out(t % 2)

    @pl.when(wid == 0)
    def _():
        pltpu.sync_copy(emb_tail, tail_v)
        for j in range(16):
            for r in range(4):
                row = 4 * j + r
                lines[0][j, pl.ds(r * 32, 16)] = tail_v[row, pl.ds(0, 16)]
                lines[0][j, pl.ds(r * 32 + 16, 16)] = tail_v[
                    row, pl.ds(16, 16)
                ]
        pltpu.sync_copy(
            lines[0].at[pl.ds(0, 16), :],
            scr.at[pl.ds(SCR_LINES - 32, 16), :],
        )


@functools.partial(
    pl.kernel,
    out_type=jax.ShapeDtypeStruct((FIELDS, EMBED_DIM, BATCH), jnp.float32),
    mesh=_MESH,
    scratch_types=[
        [pltpu.VMEM((128,), jnp.int32) for _ in range(2)],
        [pltpu.VMEM((128,), jnp.int32) for _ in range(2)],
        [pltpu.VMEM((128,), jnp.int32) for _ in range(2)],
        [pltpu.VMEM((128, 129), jnp.float32) for _ in range(2)],
        [pltpu.VMEM((EMBED_DIM, 128), jnp.float32) for _ in range(2)],
        [pltpu.SemaphoreType.DMA for _ in range(2)],
        [pltpu.SemaphoreType.DMA for _ in range(2)],
        [pltpu.SemaphoreType.DMA for _ in range(2)],
    ],
    compiler_params=_PARAMS,
)
def _gather(scr, x_t, out_p, xv, qv, cb, rows, obuf, xsem, gsem, osem):
    wid = lax.axis_index("s") * NUM_CORES + lax.axis_index("c")
    iota16 = lax.iota(jnp.int32, 16)

    def fb(it):
        u = wid * UPW + it
        return u // BBLOCKS, u % BBLOCKS

    def start_x(it, b):
        f, bb = fb(it)
        b0 = pl.multiple_of(bb * 128, 128)
        pltpu.async_copy(x_t.at[f, pl.ds(b0, 128)], xv[b], xsem[b])

    def wait_x(b):
        pltpu.make_async_copy(
            x_t.at[0, pl.ds(0, 128)], xv[b], xsem[b]
        ).wait()

    def prep(b):
        for g in range(8):
            v = xv[b][pl.ds(16 * g, 16)]
            qv[b][pl.ds(16 * g, 16)] = lax.shift_right_logical(v, 2)
            cb[b][pl.ds(16 * g, 16)] = lax.shift_left(v & 3, 5)

    def start_g(b):
        pltpu.async_copy(scr.at[qv[b]], rows[b].at[:, pl.ds(0, 128)], gsem[b])

    def wait_g(b):
        pltpu.make_async_copy(
            scr.at[qv[b]], rows[b].at[:, pl.ds(0, 128)], gsem[b]
        ).wait()

    def assemble(b):
        def eblock(eh, carry):
            for g in range(8):
                colb = cb[b][pl.ds(16 * g, 16)]
                rowv = iota16 + 16 * g
                for de in range(8):
                    e = eh * 8 + de
                    obuf[b][e, pl.ds(16 * g, 16)] = plsc.load_gather(
                        rows[b], [rowv, colb + e]
                    )
            return carry

        lax.fori_loop(0, EMBED_DIM // 8, eblock, 0)

    def start_out(it, b):
        f, bb = fb(it)
        b0 = pl.multiple_of(bb * 128, 128)
        pltpu.async_copy(obuf[b], out_p.at[f, :, pl.ds(b0, 128)], osem[b])

    def wait_out(b):
        pltpu.make_async_copy(
            obuf[b], out_p.at[0, :, pl.ds(0, 128)], osem[b]
        ).wait()

    start_x(0, 0)
    start_x(1, 1)
    wait_x(0)
    prep(0)
    start_g(0)

    def body(step, carry):
        for half in range(2):
            it = 2 * step + half
            b = half
            nb = 1 - b

            @pl.when(it + 1 < UPW)
            def _():
                wait_x(nb)
                prep(nb)
                start_g(nb)

            @pl.when(it + 2 < UPW)
            def _():
                start_x(it + 2, b)

            wait_g(b)

            @pl.when(it >= 2)
            def _():
                wait_out(b)

            assemble(b)
            start_out(it, b)
        return carry

    lax.fori_loop(0, UPW // 2, body, 0)
    wait_out(0)
    wait_out(1)


def kernel(embedding, x):
    emb_t = embedding.T
    emb_tail = embedding[VOCAB - 64:, :]
    x_t = x.T.astype(jnp.int32)
    scr = _convert(emb_t, emb_tail)
    out_p = _gather(scr, x_t)
    return out_p.transpose(2, 0, 1)


# speedup vs baseline: 1.4613x; 1.4613x over previous
import functools

import jax
import jax.numpy as jnp
from jax import lax
from jax.experimental import pallas as pl
from jax.experimental.pallas import tpu as pltpu
from jax.experimental.pallas import tpu_sc as plsc

VOCAB = 1000000
EMBED_DIM = 32
BATCH = 16384
FIELDS = 26
TOTAL = BATCH * FIELDS

NUM_CORES = 2
NUM_SUBCORES = 16
NUM_WORKERS = NUM_CORES * NUM_SUBCORES
PER_WORKER = TOTAL // NUM_WORKERS
CHUNK = 1024
NUM_CHUNKS = PER_WORKER // CHUNK
NBUF = 3

TC_BC = 2048
TC_GRID = (VOCAB + TC_BC - 1) // TC_BC

assert PER_WORKER * NUM_WORKERS == TOTAL
assert CHUNK * NUM_CHUNKS == PER_WORKER


def _tc_transpose_body(emb_t_ref, out_ref):
    blk = emb_t_ref[...]
    t = jnp.transpose(blk).reshape(TC_BC // 4, 4, EMBED_DIM)
    packed = jnp.concatenate([t[:, r, :] for r in range(4)], axis=1)
    out_ref[...] = packed.reshape(TC_BC * EMBED_DIM)


_tc_transpose = pl.pallas_call(
    _tc_transpose_body,
    grid=(TC_GRID,),
    in_specs=[
        pl.BlockSpec((EMBED_DIM, TC_BC), lambda i: (0, i)),
    ],
    out_specs=pl.BlockSpec((TC_BC * EMBED_DIM,), lambda i: (i,)),
    out_shape=jax.ShapeDtypeStruct((VOCAB * EMBED_DIM,), jnp.float32),
)


@functools.partial(
    pl.kernel,
    out_type=jax.ShapeDtypeStruct((TOTAL, EMBED_DIM), jnp.float32),
    mesh=plsc.VectorSubcoreMesh(core_axis_name="c", subcore_axis_name="s"),
    scratch_types=[
        [pltpu.VMEM((CHUNK,), jnp.int32) for _ in range(NBUF)],
        [pltpu.VMEM((CHUNK, EMBED_DIM), jnp.float32) for _ in range(NBUF)],
        [pltpu.SemaphoreType.DMA for _ in range(NBUF)],
        [pltpu.SemaphoreType.DMA for _ in range(NBUF)],
    ],
    compiler_params=pltpu.CompilerParams(use_tc_tiling_on_sc=False),
)
def _gather_rows(table_hbm, idx_hbm, out_hbm, idx_v, rows_v, gsem, ssem):
    wid = lax.axis_index("s") * NUM_CORES + lax.axis_index("c")
    base = wid * PER_WORKER

    gathers = [None] * NUM_CHUNKS
    stores = [None] * NUM_CHUNKS

    def start_gather(i):
        b = i % NBUF
        off = pl.multiple_of(base + i * CHUNK, 8)
        pltpu.sync_copy(idx_hbm.at[pl.ds(off, CHUNK)], idx_v[b])
        gathers[i] = pltpu.async_copy(table_hbm.at[idx_v[b]], rows_v[b], gsem[b])

    start_gather(0)
    start_gather(1)
    for i in range(NUM_CHUNKS):
        b = i % NBUF
        j = i + 2
        if j < NUM_CHUNKS:
            if j - NBUF >= 0:
                stores[j - NBUF].wait()
            start_gather(j)
        gathers[i].wait()
        off = pl.multiple_of(base + i * CHUNK, 8)
        stores[i] = pltpu.async_copy(rows_v[b], out_hbm.at[pl.ds(off, CHUNK)], ssem[b])
    for i in range(max(0, NUM_CHUNKS - NBUF), NUM_CHUNKS):
        stores[i].wait()


def kernel(embedding, x):
    emb_t = embedding.T
    table = _tc_transpose(emb_t).reshape(VOCAB, EMBED_DIM)
    idx = x.reshape(TOTAL).astype(jnp.int32)
    out = _gather_rows(table, idx)
    return out.reshape(BATCH, FIELDS, EMBED_DIM)
